# baseline (speedup 1.0000x reference)
import functools

import jax
import jax.numpy as jnp
from jax import lax
from jax.experimental import pallas as pl
from jax.experimental.pallas import tpu as pltpu
from jax.experimental.pallas import tpu_sc as plsc

_VOCAB = 1000000
_EMSIZE = 64
_BATCH = 4096
_SEQLEN = 200

_N = _BATCH * _SEQLEN
_CHUNK = 128
_NROWS = _N // _CHUNK

_info = plsc.get_sparse_core_info()
_NC, _NS = _info.num_cores, _info.num_subcores
_NW = _NC * _NS
_CPW = _NROWS // _NW


def _make_gather():
    mesh = plsc.VectorSubcoreMesh(core_axis_name="c", subcore_axis_name="s")

    @functools.partial(
        pl.kernel,
        mesh=mesh,
        out_type=jax.ShapeDtypeStruct((_N, _EMSIZE), jnp.float32),
        scratch_types=[
            pltpu.VMEM((_CPW, _CHUNK), jnp.int32),
            pltpu.VMEM((_CHUNK, _EMSIZE), jnp.float32),
            pltpu.SemaphoreType.DMA,
        ],
        compiler_params=pltpu.CompilerParams(use_tc_tiling_on_sc=False),
    )
    def gather_kernel(table_hbm, idx_hbm, out_hbm, idx_v, buf, sem):
        wid = lax.axis_index("s") * _NC + lax.axis_index("c")
        row0 = wid * _CPW
        pltpu.sync_copy(idx_hbm.at[pl.ds(row0, _CPW)], idx_v)

        def body(g, carry):
            pltpu.async_copy(table_hbm.at[idx_v.at[g]], buf, sem).wait()
            base = (row0 + g) * _CHUNK
            pltpu.sync_copy(buf, out_hbm.at[pl.ds(base, _CHUNK)])
            return carry

        lax.fori_loop(0, _CPW, body, 0)

    return gather_kernel


_gather = _make_gather()


def kernel(sequence, table):
    idx = sequence.astype(jnp.int32).reshape(_NROWS, _CHUNK)
    out = _gather(table, idx)
    return out.reshape(_BATCH, _SEQLEN, _EMSIZE)


# speedup vs baseline: 1.1127x; 1.1127x over previous
import functools

import jax
import jax.numpy as jnp
from jax import lax
from jax.experimental import pallas as pl
from jax.experimental.pallas import tpu as pltpu
from jax.experimental.pallas import tpu_sc as plsc

_VOCAB = 1000000
_EMSIZE = 64
_BATCH = 4096
_SEQLEN = 200

_N = _BATCH * _SEQLEN
_CHUNK = 128
_NROWS = _N // _CHUNK

_info = plsc.get_sparse_core_info()
_NC, _NS = _info.num_cores, _info.num_subcores
_NW = _NC * _NS
_CPW = _NROWS // _NW


_NB = 4


def _make_gather():
    mesh = plsc.VectorSubcoreMesh(core_axis_name="c", subcore_axis_name="s")

    @functools.partial(
        pl.kernel,
        mesh=mesh,
        out_type=jax.ShapeDtypeStruct((_N, _EMSIZE), jnp.float32),
        scratch_types=[
            pltpu.VMEM((_CPW, _CHUNK), jnp.int32),
            [pltpu.VMEM((_CHUNK, _EMSIZE), jnp.float32) for _ in range(_NB)],
            pltpu.SemaphoreType.DMA((_NB,)),
            pltpu.SemaphoreType.DMA((_NB,)),
        ],
        compiler_params=pltpu.CompilerParams(use_tc_tiling_on_sc=False),
    )
    def gather_kernel(table_hbm, idx_hbm, out_hbm, idx_v, bufs, gsem, wsem):
        wid = lax.axis_index("s") * _NC + lax.axis_index("c")
        row0 = wid * _CPW
        pltpu.sync_copy(idx_hbm.at[pl.ds(row0, _CPW)], idx_v)

        for b in range(_NB):
            pltpu.async_copy(table_hbm.at[idx_v.at[b]], bufs[b], gsem.at[b])

        def body(i, carry):
            for b in range(_NB):
                g = i * _NB + b
                pltpu.make_async_copy(
                    table_hbm.at[idx_v.at[g]], bufs[b], gsem.at[b]
                ).wait()
                base = (row0 + g) * _CHUNK
                pltpu.async_copy(
                    bufs[b], out_hbm.at[pl.ds(base, _CHUNK)], wsem.at[b]
                )
            for b in range(_NB):
                gnext = jnp.minimum((i + 1) * _NB + b, _CPW - 1)
                pltpu.make_async_copy(
                    bufs[b], out_hbm.at[pl.ds(0, _CHUNK)], wsem.at[b]
                ).wait()
                pltpu.async_copy(table_hbm.at[idx_v.at[gnext]], bufs[b], gsem.at[b])
            return carry

        lax.fori_loop(0, _CPW // _NB, body, 0)

        for b in range(_NB):
            pltpu.make_async_copy(
                table_hbm.at[idx_v.at[0]], bufs[b], gsem.at[b]
            ).wait()

    return gather_kernel


_gather = _make_gather()


def kernel(sequence, table):
    idx = sequence.astype(jnp.int32).reshape(_NROWS, _CHUNK)
    out = _gather(table, idx)
    return out.reshape(_BATCH, _SEQLEN, _EMSIZE)
